# Initial kernel scaffold; baseline (speedup 1.0000x reference)
import jax
import jax.numpy as jnp
from jax.experimental import pallas as pl


def kernel(x, edge_index, batch, c1_Wrel, c1_brel, c1_Wroot, c2_Wrel, c2_brel, c2_Wroot, c3_Wrel, c3_brel, c3_Wroot, lin1_W, lin1_b, lin2_W, lin2_b):
    raise NotImplementedError("write your pallas kernel here")


# speedup vs baseline: 4.5816x; 4.5816x over previous
import functools

import jax
import jax.numpy as jnp
from jax import lax
from jax.experimental import pallas as pl
from jax.experimental.pallas import tpu as pltpu
from jax.experimental.pallas import tpu_sc as plsc

_N = 10000
_E = 320000
_DIN = 128
_HID = 256
_NCLS = 10
_NG = 64

_NC, _NS, _L = 2, 16, 16
_NW = _NC * _NS
_CPT = _NG // _NW
_SLAB = _N * _CPT
_CHUNK = 6400
_NCHUNK = _E // _CHUNK
_IPC = _CHUNK // _L


def _build_sc(interpret=False):
    mesh = plsc.VectorSubcoreMesh(core_axis_name="c", subcore_axis_name="s",
                                  num_cores=_NC, num_subcores=_NS)
    out_t = [jax.ShapeDtypeStruct((_NW, _SLAB), jnp.float32) for _ in range(3)]

    @functools.partial(
        pl.kernel,
        out_type=out_t,
        mesh=mesh,
        interpret=interpret,
        compiler_params=pltpu.CompilerParams(needs_layout_passes=False),
        scratch_types=[
            pltpu.VMEM((_SLAB,), jnp.float32),
            pltpu.VMEM((_SLAB,), jnp.float32),
            pltpu.VMEM((_SLAB,), jnp.float32),
            pltpu.VMEM((_N,), jnp.int32),
            pltpu.VMEM((_CHUNK,), jnp.int32),
            pltpu.VMEM((_CHUNK,), jnp.int32),
        ],
    )
    def sc_mchain(edges, batch, m1o, m2o, m3o, m1v, m2v, m3v, batv, srcv, dstv):
        wid = lax.axis_index("c") * _NS + lax.axis_index("s")
        colbase = wid * _CPT

        for ref in (m1v, m2v, m3v):
            def zb(i, _, ref=ref):
                ref[pl.ds(i * _L, _L)] = jnp.zeros((_L,), jnp.float32)
                return 0
            lax.fori_loop(0, _SLAB // _L, zb, 0)
        pltpu.sync_copy(batch, batv)

        def sweep(body):
            def cb(ci, _):
                off = ci * _CHUNK
                pltpu.sync_copy(edges.at[0, pl.ds(off, _CHUNK)], srcv)
                pltpu.sync_copy(edges.at[1, pl.ds(off, _CHUNK)], dstv)

                def ib(i, _):
                    s16 = srcv[pl.ds(i * _L, _L)]
                    d16 = dstv[pl.ds(i * _L, _L)]
                    body(s16, d16)
                    return 0
                lax.fori_loop(0, _IPC, ib, 0)
                return 0
            lax.fori_loop(0, _NCHUNK, cb, 0)

        ones = jnp.full((_L,), 1.0, jnp.float32)

        def pass_a(s16, d16):
            gcol = plsc.load_gather(batv, [d16])
            loc = gcol - colbase
            msk = (loc >= 0) & (loc < _CPT)
            locc = jnp.clip(loc, 0, _CPT - 1)
            plsc.addupdate_scatter(m1v, [s16 * _CPT + locc], ones, mask=msk)
        sweep(pass_a)

        def prop(prev, nxt):
            def body(s16, d16):
                bd = d16 * _CPT
                bs = s16 * _CPT
                for c in range(_CPT):
                    v = plsc.load_gather(prev, [bd + c])
                    plsc.addupdate_scatter(nxt, [bs + c], v)
            sweep(body)
        prop(m1v, m2v)
        prop(m2v, m3v)

        pltpu.sync_copy(m1v, m1o.at[wid])
        pltpu.sync_copy(m2v, m2o.at[wid])
        pltpu.sync_copy(m3v, m3o.at[wid])

    return sc_mchain


def _tc_body(x_ref, m1_ref, m2_ref, m3_ref, bat_ref,
             w1r_ref, w1o_ref, b1_ref, w2r_ref, w2o_ref, b2_ref,
             w3r_ref, w3o_ref, b3_ref, l1w_ref, l1b_ref, l2w_ref, l2b_ref,
             out_ref):
    def mm(a, b):
        return lax.dot_general(a, b, (((1,), (0,)), ((), ())),
                               precision=lax.Precision.HIGHEST,
                               preferred_element_type=jnp.float32)

    x = x_ref[:]
    m0 = (bat_ref[:] == lax.broadcasted_iota(jnp.int32, (_NG, _N), 0)
          ).astype(jnp.float32)
    m1 = m1_ref[:]
    m2 = m2_ref[:]
    m3 = m3_ref[:]

    s0 = mm(m0, x)
    s1 = mm(m1, x)
    s2 = mm(m2, x)
    s3 = mm(m3, x)
    n0 = jnp.sum(m0, axis=1, keepdims=True)
    n1 = jnp.sum(m1, axis=1, keepdims=True)
    n2 = jnp.sum(m2, axis=1, keepdims=True)

    w1r = w1r_ref[:]; w1o = w1o_ref[:]; b1 = b1_ref[:]
    w2r = w2r_ref[:]; w2o = w2o_ref[:]; b2 = b2_ref[:]
    w3r = w3r_ref[:]; w3o = w3o_ref[:]; b3 = b3_ref[:]

    t2 = mm(s3, w1r) + mm(s2, w1o) + n2 * b1
    t1 = mm(s2, w1r) + mm(s1, w1o) + n1 * b1
    t0 = mm(s1, w1r) + mm(s0, w1o) + n0 * b1
    a3 = mm(w2r, w3r)
    amid = mm(w2o, w3r) + mm(w2r, w3o)
    alo = mm(w2o, w3o)
    g = (mm(t2, a3) + mm(t1, amid) + mm(t0, alo)
         + n1 * mm(b2, w3r) + n0 * (mm(b2, w3o) + b3))

    z1 = jnp.maximum(mm(g, l1w_ref[:]) + l1b_ref[:], 0.0)
    z = mm(z1, l2w_ref[:]) + l2b_ref[:]
    zc = z - jnp.max(z, axis=1, keepdims=True)
    out_ref[:] = zc - jnp.log(jnp.sum(jnp.exp(zc), axis=1, keepdims=True))


def _build_tc(interpret=False):
    return pl.pallas_call(
        _tc_body,
        out_shape=jax.ShapeDtypeStruct((_NG, _NCLS), jnp.float32),
        interpret=interpret,
    )


@functools.lru_cache(maxsize=None)
def _get_sc():
    return _build_sc()


@functools.lru_cache(maxsize=None)
def _get_tc():
    return _build_tc()


def kernel(x, edge_index, batch,
           c1_Wrel, c1_brel, c1_Wroot,
           c2_Wrel, c2_brel, c2_Wroot,
           c3_Wrel, c3_brel, c3_Wroot,
           lin1_W, lin1_b, lin2_W, lin2_b):
    ei = edge_index.astype(jnp.int32)
    bat = batch.astype(jnp.int32)
    m1t, m2t, m3t = _get_sc()(ei, bat)

    def m_t(t):
        return t.reshape(_NW, _N, _CPT).transpose(0, 2, 1).reshape(_NG, _N)

    return _get_tc()(x, m_t(m1t), m_t(m2t), m_t(m3t), bat.reshape(1, _N),
               c1_Wrel, c1_Wroot, c1_brel.reshape(1, _HID),
               c2_Wrel, c2_Wroot, c2_brel.reshape(1, _HID),
               c3_Wrel, c3_Wroot, c3_brel.reshape(1, _HID),
               lin1_W, lin1_b.reshape(1, _HID),
               lin2_W, lin2_b.reshape(1, _NCLS))


# speedup vs baseline: 21.5133x; 4.6956x over previous
import functools

import jax
import jax.numpy as jnp
from jax import lax
from jax.experimental import pallas as pl
from jax.experimental.pallas import tpu as pltpu
from jax.experimental.pallas import tpu_sc as plsc

_N = 10000
_E = 320000
_DIN = 128
_HID = 256
_NCLS = 10
_NG = 64

_NC, _NS, _L = 2, 16, 16
_NW = _NC * _NS
_EPT = _E // _NW
_CH = 400
_NCH = _EPT // _CH
_RPT = _N // _NS


def _mesh():
    return plsc.VectorSubcoreMesh(core_axis_name="c", subcore_axis_name="s",
                                  num_cores=_NC, num_subcores=_NS)


def _build_sweep_a(interpret=False):
    @functools.partial(
        pl.kernel,
        out_type=jax.ShapeDtypeStruct((_NC, _N * _NG), jnp.float32),
        mesh=_mesh(),
        interpret=interpret,
        compiler_params=pltpu.CompilerParams(needs_layout_passes=False, use_tc_tiling_on_sc=False),
        scratch_types=[
            pltpu.VMEM_SHARED((_N * _NG,), jnp.float32),
            pltpu.VMEM((_EPT,), jnp.int32),
            pltpu.VMEM((_EPT,), jnp.int32),
            pltpu.VMEM((_N,), jnp.int32),
            pltpu.VMEM((_NCH, _CH), jnp.int32),
            pltpu.VMEM((_CH,), jnp.float32),
        ],
    )
    def sweep_a(src_h, dst_h, batch_h, zeros_h, out_h, acc, srcv, dstv, batv,
                idxv, ones):
        cid = lax.axis_index("c")
        sid = lax.axis_index("s")
        ebase = (cid * _NS + sid) * _EPT
        words = (_N * _NG) // _NS

        pltpu.sync_copy(zeros_h.at[pl.ds(sid * words, words)],
                        acc.at[pl.ds(sid * words, words)])

        pltpu.sync_copy(src_h.at[pl.ds(ebase, _EPT)], srcv)
        pltpu.sync_copy(dst_h.at[pl.ds(ebase, _EPT)], dstv)
        pltpu.sync_copy(batch_h, batv)

        def ob(i, _):
            ones[pl.ds(i * _L, _L)] = jnp.full((_L,), 1.0, jnp.float32)
            return 0
        lax.fori_loop(0, _CH // _L, ob, 0)

        for ci in range(_NCH):
            def ib(j, _, ci=ci):
                o = ci * _CH + j * _L
                s16 = srcv[pl.ds(o, _L)]
                d16 = dstv[pl.ds(o, _L)]
                b16 = plsc.load_gather(batv, [d16])
                idxv[ci, pl.ds(j * _L, _L)] = s16 * _NG + b16
                return 0
            lax.fori_loop(0, _CH // _L, ib, 0)

        plsc.subcore_barrier()
        for ci in range(_NCH):
            pltpu.sync_copy(ones, acc.at[idxv.at[ci]], add=True)
        plsc.subcore_barrier()

        pltpu.sync_copy(acc.at[pl.ds(sid * words, words)],
                        out_h.at[cid, pl.ds(sid * words, words)])

    return sweep_a


def _build_sweep_bc(interpret=False):
    @functools.partial(
        pl.kernel,
        out_type=jax.ShapeDtypeStruct((_NC, _N, _NG), jnp.float32),
        mesh=_mesh(),
        interpret=interpret,
        compiler_params=pltpu.CompilerParams(needs_layout_passes=False, use_tc_tiling_on_sc=False),
        scratch_types=[
            pltpu.VMEM_SHARED((_N, _NG), jnp.float32),
            pltpu.VMEM((_NCH, _CH), jnp.int32),
            pltpu.VMEM((_NCH, _CH), jnp.int32),
            pltpu.VMEM((2, _CH, _NG), jnp.float32),
            pltpu.SemaphoreType.DMA,
            pltpu.SemaphoreType.DMA,
        ],
    )
    def sweep_bc(src_h, dst_h, table_h, zeros_h, out_h, acc, sidx, didx, rows,
                 sem0, sem1):
        cid = lax.axis_index("c")
        sid = lax.axis_index("s")
        ebase = (cid * _NS + sid) * _EPT

        pltpu.sync_copy(zeros_h.at[pl.ds(sid * _RPT, _RPT)],
                        acc.at[pl.ds(sid * _RPT, _RPT)])

        for ci in range(_NCH):
            pltpu.sync_copy(src_h.at[pl.ds(ebase + ci * _CH, _CH)],
                            sidx.at[ci])
            pltpu.sync_copy(dst_h.at[pl.ds(ebase + ci * _CH, _CH)],
                            didx.at[ci])

        plsc.subcore_barrier()
        sems = (sem0, sem1)
        pltpu.async_copy(table_h.at[didx.at[0]], rows.at[0], sems[0])
        for ci in range(_NCH):
            b = ci % 2
            pltpu.make_async_copy(table_h.at[didx.at[ci]], rows.at[b],
                                  sems[b]).wait()
            if ci + 1 < _NCH:
                pltpu.async_copy(table_h.at[didx.at[ci + 1]],
                                 rows.at[1 - b], sems[1 - b])
            pltpu.sync_copy(rows.at[b], acc.at[sidx.at[ci]], add=True)
        plsc.subcore_barrier()

        pltpu.sync_copy(acc.at[pl.ds(sid * _RPT, _RPT)],
                        out_h.at[cid, pl.ds(sid * _RPT, _RPT)])

    return sweep_bc


def _combine_body(p_ref, o_ref):
    o_ref[:] = p_ref[0] + p_ref[1]


def _build_combine(interpret=False):
    return pl.pallas_call(
        _combine_body,
        out_shape=jax.ShapeDtypeStruct((_N, _NG), jnp.float32),
        interpret=interpret,
    )


def _tc_body(x_ref, m1_ref, m2_ref, p3_ref, bat_ref,
             w1r_ref, w1o_ref, b1_ref, w2r_ref, w2o_ref, b2_ref,
             w3r_ref, w3o_ref, b3_ref, l1w_ref, l1b_ref, l2w_ref, l2b_ref,
             out_ref):
    def mm(a, b):
        return lax.dot_general(a, b, (((1,), (0,)), ((), ())),
                               precision=lax.Precision.HIGHEST,
                               preferred_element_type=jnp.float32)

    def tmm(a, b):
        return lax.dot_general(a, b, (((0,), (0,)), ((), ())),
                               precision=lax.Precision.HIGHEST,
                               preferred_element_type=jnp.float32)

    x = x_ref[:]
    m0t = (bat_ref[:] == lax.broadcasted_iota(jnp.int32, (_NG, _N), 0)
           ).astype(jnp.float32)
    m1 = m1_ref[:]
    m2 = m2_ref[:]
    m3 = p3_ref[0] + p3_ref[1]

    s0 = mm(m0t, x)
    s1 = tmm(m1, x)
    s2 = tmm(m2, x)
    s3 = tmm(m3, x)
    n0 = jnp.sum(m0t, axis=1, keepdims=True)
    n1 = jnp.sum(m1, axis=0, keepdims=True).reshape(_NG, 1)
    n2 = jnp.sum(m2, axis=0, keepdims=True).reshape(_NG, 1)

    w1r = w1r_ref[:]; w1o = w1o_ref[:]; b1 = b1_ref[:]
    w2r = w2r_ref[:]; w2o = w2o_ref[:]; b2 = b2_ref[:]
    w3r = w3r_ref[:]; w3o = w3o_ref[:]; b3 = b3_ref[:]

    t2 = mm(s3, w1r) + mm(s2, w1o) + n2 * b1
    t1 = mm(s2, w1r) + mm(s1, w1o) + n1 * b1
    t0 = mm(s1, w1r) + mm(s0, w1o) + n0 * b1
    a3 = mm(w2r, w3r)
    amid = mm(w2o, w3r) + mm(w2r, w3o)
    alo = mm(w2o, w3o)
    g = (mm(t2, a3) + mm(t1, amid) + mm(t0, alo)
         + n1 * mm(b2, w3r) + n0 * (mm(b2, w3o) + b3))

    z1 = jnp.maximum(mm(g, l1w_ref[:]) + l1b_ref[:], 0.0)
    z = mm(z1, l2w_ref[:]) + l2b_ref[:]
    zc = z - jnp.max(z, axis=1, keepdims=True)
    out_ref[:] = zc - jnp.log(jnp.sum(jnp.exp(zc), axis=1, keepdims=True))


def _build_tc(interpret=False):
    return pl.pallas_call(
        _tc_body,
        out_shape=jax.ShapeDtypeStruct((_NG, _NCLS), jnp.float32),
        interpret=interpret,
    )


@functools.lru_cache(maxsize=None)
def _get_kernels():
    return (_build_sweep_a(), _build_sweep_bc(), _build_combine(),
            _build_tc())


def kernel(x, edge_index, batch,
           c1_Wrel, c1_brel, c1_Wroot,
           c2_Wrel, c2_brel, c2_Wroot,
           c3_Wrel, c3_brel, c3_Wroot,
           lin1_W, lin1_b, lin2_W, lin2_b):
    ei = edge_index.astype(jnp.int32)
    bat = batch.astype(jnp.int32)
    src, dst = ei[0], ei[1]
    sweep_a, sweep_bc, combine, tc = _get_kernels()

    zflat = jnp.zeros((_N * _NG,), jnp.float32)
    p1 = sweep_a(src, dst, bat, zflat)
    m1 = combine(p1.reshape(_NC, _N, _NG))
    p2 = sweep_bc(src, dst, m1, zflat.reshape(_N, _NG))
    m2 = combine(p2)
    p3 = sweep_bc(src, dst, m2, zflat.reshape(_N, _NG))

    return tc(x, m1, m2, p3, bat.reshape(1, _N),
              c1_Wrel, c1_Wroot, c1_brel.reshape(1, _HID),
              c2_Wrel, c2_Wroot, c2_brel.reshape(1, _HID),
              c3_Wrel, c3_Wroot, c3_brel.reshape(1, _HID),
              lin1_W, lin1_b.reshape(1, _HID),
              lin2_W, lin2_b.reshape(1, _NCLS))


# speedup vs baseline: 25.9600x; 1.2067x over previous
import functools

import jax
import jax.numpy as jnp
from jax import lax
from jax.experimental import pallas as pl
from jax.experimental.pallas import tpu as pltpu
from jax.experimental.pallas import tpu_sc as plsc

_N = 10000
_E = 320000
_DIN = 128
_HID = 256
_NCLS = 10
_NG = 64

_NC, _NS, _L = 2, 16, 16
_NW = _NC * _NS
_EPT = _E // _NW
_CH = 400
_NCH = _EPT // _CH
_RPT = _N // _NS


def _mesh():
    return plsc.VectorSubcoreMesh(core_axis_name="c", subcore_axis_name="s",
                                  num_cores=_NC, num_subcores=_NS)


def _build_sweep_a(interpret=False):
    @functools.partial(
        pl.kernel,
        out_type=jax.ShapeDtypeStruct((_NC, _N * _NG), jnp.float32),
        mesh=_mesh(),
        interpret=interpret,
        compiler_params=pltpu.CompilerParams(needs_layout_passes=False, use_tc_tiling_on_sc=False),
        scratch_types=[
            pltpu.VMEM_SHARED((_N * _NG,), jnp.float32),
            pltpu.VMEM((_EPT,), jnp.int32),
            pltpu.VMEM((_EPT,), jnp.int32),
            pltpu.VMEM((_N,), jnp.int32),
            pltpu.VMEM((_NCH, _CH), jnp.int32),
            pltpu.VMEM((_CH,), jnp.float32),
        ],
    )
    def sweep_a(src_h, dst_h, batch_h, zeros_h, out_h, acc, srcv, dstv, batv,
                idxv, ones):
        cid = lax.axis_index("c")
        sid = lax.axis_index("s")
        ebase = (cid * _NS + sid) * _EPT
        words = (_N * _NG) // _NS

        pltpu.sync_copy(zeros_h.at[pl.ds(sid * words, words)],
                        acc.at[pl.ds(sid * words, words)])

        pltpu.sync_copy(src_h.at[pl.ds(ebase, _EPT)], srcv)
        pltpu.sync_copy(dst_h.at[pl.ds(ebase, _EPT)], dstv)
        pltpu.sync_copy(batch_h, batv)

        def ob(i, _):
            ones[pl.ds(i * _L, _L)] = jnp.full((_L,), 1.0, jnp.float32)
            return 0
        lax.fori_loop(0, _CH // _L, ob, 0)

        for ci in range(_NCH):
            def ib(j, _, ci=ci):
                o = ci * _CH + j * _L
                s16 = srcv[pl.ds(o, _L)]
                d16 = dstv[pl.ds(o, _L)]
                b16 = plsc.load_gather(batv, [d16])
                idxv[ci, pl.ds(j * _L, _L)] = s16 * _NG + b16
                return 0
            lax.fori_loop(0, _CH // _L, ib, 0)

        plsc.subcore_barrier()
        for ci in range(_NCH):
            pltpu.sync_copy(ones, acc.at[idxv.at[ci]], add=True)
        plsc.subcore_barrier()

        pltpu.sync_copy(acc.at[pl.ds(sid * words, words)],
                        out_h.at[cid, pl.ds(sid * words, words)])

    return sweep_a


def _build_sweep_bc(interpret=False):
    @functools.partial(
        pl.kernel,
        out_type=jax.ShapeDtypeStruct((_NC, _N, _NG), jnp.float32),
        mesh=_mesh(),
        interpret=interpret,
        compiler_params=pltpu.CompilerParams(needs_layout_passes=False, use_tc_tiling_on_sc=False),
        scratch_types=[
            pltpu.VMEM_SHARED((_N, _NG), jnp.float32),
            pltpu.VMEM((3, _CH), jnp.int32),
            pltpu.VMEM((3, _CH), jnp.int32),
            pltpu.VMEM((3, _CH, _NG), jnp.float32),
            pltpu.SemaphoreType.DMA,
            pltpu.SemaphoreType.DMA,
            pltpu.SemaphoreType.DMA,
            pltpu.SemaphoreType.DMA,
            pltpu.SemaphoreType.DMA,
            pltpu.SemaphoreType.DMA,
        ],
    )
    def sweep_bc(src_h, dst_h, table_h, zeros_h, out_h, acc, sidx, didx, rows,
                 g0, g1, g2, s0, s1, s2):
        cid = lax.axis_index("c")
        sid = lax.axis_index("s")
        ebase = (cid * _NS + sid) * _EPT
        gsem = (g0, g1, g2)
        ssem = (s0, s1, s2)

        pltpu.sync_copy(zeros_h.at[pl.ds(sid * _RPT, _RPT)],
                        acc.at[pl.ds(sid * _RPT, _RPT)])
        plsc.subcore_barrier()

        def fill(ci):
            b = ci % 3
            pltpu.sync_copy(dst_h.at[pl.ds(ebase + ci * _CH, _CH)],
                            didx.at[b])
            pltpu.sync_copy(src_h.at[pl.ds(ebase + ci * _CH, _CH)],
                            sidx.at[b])
            pltpu.async_copy(table_h.at[didx.at[b]], rows.at[b], gsem[b])

        def wait_scatter(b):
            pltpu.make_async_copy(rows.at[b], acc.at[sidx.at[b]],
                                  ssem[b]).wait()

        fill(0)
        if _NCH > 1:
            fill(1)
        for ci in range(_NCH):
            b = ci % 3
            pltpu.make_async_copy(table_h.at[didx.at[b]], rows.at[b],
                                  gsem[b]).wait()
            pltpu.async_copy(rows.at[b], acc.at[sidx.at[b]], ssem[b],
                             add=True)
            nxt = ci + 2
            if nxt < _NCH:
                bb = nxt % 3
                if ci >= 1:
                    wait_scatter(bb)
                fill(nxt)
        for k in range(max(0, _NCH - 3), _NCH):
            wait_scatter(k % 3)
        plsc.subcore_barrier()

        pltpu.sync_copy(acc.at[pl.ds(sid * _RPT, _RPT)],
                        out_h.at[cid, pl.ds(sid * _RPT, _RPT)])

    return sweep_bc


def _combine_body(p_ref, o_ref):
    o_ref[:] = p_ref[0] + p_ref[1]


def _build_combine(interpret=False):
    return pl.pallas_call(
        _combine_body,
        out_shape=jax.ShapeDtypeStruct((_N, _NG), jnp.float32),
        interpret=interpret,
    )


def _tc_body(x_ref, m1_ref, m2_ref, p3_ref, bat_ref,
             w1r_ref, w1o_ref, b1_ref, w2r_ref, w2o_ref, b2_ref,
             w3r_ref, w3o_ref, b3_ref, l1w_ref, l1b_ref, l2w_ref, l2b_ref,
             out_ref):
    def mm(a, b):
        return lax.dot_general(a, b, (((1,), (0,)), ((), ())),
                               precision=lax.Precision.HIGHEST,
                               preferred_element_type=jnp.float32)

    def tmm(a, b):
        return lax.dot_general(a, b, (((0,), (0,)), ((), ())),
                               precision=lax.Precision.HIGHEST,
                               preferred_element_type=jnp.float32)

    x = x_ref[:]
    m0t = (bat_ref[:] == lax.broadcasted_iota(jnp.int32, (_NG, _N), 0)
           ).astype(jnp.float32)
    m1 = m1_ref[:]
    m2 = m2_ref[:]
    m3 = p3_ref[0] + p3_ref[1]

    s0 = mm(m0t, x)
    s1 = tmm(m1, x)
    s2 = tmm(m2, x)
    s3 = tmm(m3, x)
    n0 = jnp.sum(m0t, axis=1, keepdims=True)
    n1 = jnp.sum(m1, axis=0, keepdims=True).reshape(_NG, 1)
    n2 = jnp.sum(m2, axis=0, keepdims=True).reshape(_NG, 1)

    w1r = w1r_ref[:]; w1o = w1o_ref[:]; b1 = b1_ref[:]
    w2r = w2r_ref[:]; w2o = w2o_ref[:]; b2 = b2_ref[:]
    w3r = w3r_ref[:]; w3o = w3o_ref[:]; b3 = b3_ref[:]

    t2 = mm(s3, w1r) + mm(s2, w1o) + n2 * b1
    t1 = mm(s2, w1r) + mm(s1, w1o) + n1 * b1
    t0 = mm(s1, w1r) + mm(s0, w1o) + n0 * b1
    a3 = mm(w2r, w3r)
    amid = mm(w2o, w3r) + mm(w2r, w3o)
    alo = mm(w2o, w3o)
    g = (mm(t2, a3) + mm(t1, amid) + mm(t0, alo)
         + n1 * mm(b2, w3r) + n0 * (mm(b2, w3o) + b3))

    z1 = jnp.maximum(mm(g, l1w_ref[:]) + l1b_ref[:], 0.0)
    z = mm(z1, l2w_ref[:]) + l2b_ref[:]
    zc = z - jnp.max(z, axis=1, keepdims=True)
    out_ref[:] = zc - jnp.log(jnp.sum(jnp.exp(zc), axis=1, keepdims=True))


def _build_tc(interpret=False):
    return pl.pallas_call(
        _tc_body,
        out_shape=jax.ShapeDtypeStruct((_NG, _NCLS), jnp.float32),
        interpret=interpret,
    )


@functools.lru_cache(maxsize=None)
def _get_kernels():
    return (_build_sweep_a(), _build_sweep_bc(), _build_combine(),
            _build_tc())


def kernel(x, edge_index, batch,
           c1_Wrel, c1_brel, c1_Wroot,
           c2_Wrel, c2_brel, c2_Wroot,
           c3_Wrel, c3_brel, c3_Wroot,
           lin1_W, lin1_b, lin2_W, lin2_b):
    ei = edge_index.astype(jnp.int32)
    bat = batch.astype(jnp.int32)
    src, dst = ei[0], ei[1]
    sweep_a, sweep_bc, combine, tc = _get_kernels()

    zflat = jnp.zeros((_N * _NG,), jnp.float32)
    p1 = sweep_a(src, dst, bat, zflat)
    m1 = combine(p1.reshape(_NC, _N, _NG))
    p2 = sweep_bc(src, dst, m1, zflat.reshape(_N, _NG))
    m2 = combine(p2)
    p3 = sweep_bc(src, dst, m2, zflat.reshape(_N, _NG))

    return tc(x, m1, m2, p3, bat.reshape(1, _N),
              c1_Wrel, c1_Wroot, c1_brel.reshape(1, _HID),
              c2_Wrel, c2_Wroot, c2_brel.reshape(1, _HID),
              c3_Wrel, c3_Wroot, c3_brel.reshape(1, _HID),
              lin1_W, lin1_b.reshape(1, _HID),
              lin2_W, lin2_b.reshape(1, _NCLS))


# speedup vs baseline: 26.0462x; 1.0033x over previous
import functools

import jax
import jax.numpy as jnp
from jax import lax
from jax.experimental import pallas as pl
from jax.experimental.pallas import tpu as pltpu
from jax.experimental.pallas import tpu_sc as plsc

_N = 10000
_E = 320000
_DIN = 128
_HID = 256
_NCLS = 10
_NG = 64

_NC, _NS, _L = 2, 16, 16
_NW = _NC * _NS
_EPT = _E // _NW
_CH = 400
_NCH = _EPT // _CH
_RPT = _N // _NS


def _mesh():
    return plsc.VectorSubcoreMesh(core_axis_name="c", subcore_axis_name="s",
                                  num_cores=_NC, num_subcores=_NS)


def _build_sweep_a(interpret=False):
    @functools.partial(
        pl.kernel,
        out_type=jax.ShapeDtypeStruct((_NC, _N * _NG), jnp.float32),
        mesh=_mesh(),
        interpret=interpret,
        compiler_params=pltpu.CompilerParams(needs_layout_passes=False, use_tc_tiling_on_sc=False),
        scratch_types=[
            pltpu.VMEM_SHARED((_N * _NG,), jnp.float32),
            pltpu.VMEM((_EPT,), jnp.int32),
            pltpu.VMEM((_EPT,), jnp.int32),
            pltpu.VMEM((_N,), jnp.int32),
            pltpu.VMEM((_NCH, _CH), jnp.int32),
            pltpu.VMEM((_CH,), jnp.float32),
        ],
    )
    def sweep_a(src_h, dst_h, batch_h, zeros_h, out_h, acc, srcv, dstv, batv,
                idxv, ones):
        cid = lax.axis_index("c")
        sid = lax.axis_index("s")
        ebase = (cid * _NS + sid) * _EPT
        words = (_N * _NG) // _NS

        pltpu.sync_copy(zeros_h.at[pl.ds(sid * words, words)],
                        acc.at[pl.ds(sid * words, words)])

        pltpu.sync_copy(src_h.at[pl.ds(ebase, _EPT)], srcv)
        pltpu.sync_copy(dst_h.at[pl.ds(ebase, _EPT)], dstv)
        pltpu.sync_copy(batch_h, batv)

        def ob(i, _):
            ones[pl.ds(i * _L, _L)] = jnp.full((_L,), 1.0, jnp.float32)
            return 0
        lax.fori_loop(0, _CH // _L, ob, 0)

        for ci in range(_NCH):
            def ib(j, _, ci=ci):
                o = ci * _CH + j * _L
                s16 = srcv[pl.ds(o, _L)]
                d16 = dstv[pl.ds(o, _L)]
                b16 = plsc.load_gather(batv, [d16])
                idxv[ci, pl.ds(j * _L, _L)] = s16 * _NG + b16
                return 0
            lax.fori_loop(0, _CH // _L, ib, 0)

        plsc.subcore_barrier()
        for ci in range(_NCH):
            pltpu.sync_copy(ones, acc.at[idxv.at[ci]], add=True)
        plsc.subcore_barrier()

        pltpu.sync_copy(acc.at[pl.ds(sid * words, words)],
                        out_h.at[cid, pl.ds(sid * words, words)])

    return sweep_a


def _build_sweep_bc(interpret=False):
    @functools.partial(
        pl.kernel,
        out_type=jax.ShapeDtypeStruct((_NC, _N, _NG), jnp.float32),
        mesh=_mesh(),
        interpret=interpret,
        compiler_params=pltpu.CompilerParams(needs_layout_passes=False, use_tc_tiling_on_sc=False),
        scratch_types=[
            pltpu.VMEM_SHARED((_N, _NG), jnp.float32),
            pltpu.VMEM((3, _CH), jnp.int32),
            pltpu.VMEM((_EPT,), jnp.int32),
            pltpu.VMEM((3, _CH, _NG), jnp.float32),
            pltpu.SemaphoreType.DMA,
            pltpu.SemaphoreType.DMA,
            pltpu.SemaphoreType.DMA,
            pltpu.SemaphoreType.DMA,
            pltpu.SemaphoreType.DMA,
            pltpu.SemaphoreType.DMA,
            pltpu.SemaphoreType.DMA,
            pltpu.SemaphoreType.DMA,
            pltpu.SemaphoreType.DMA,
        ],
    )
    def sweep_bc(src_h, dst_h, table_h, zeros_h, out_h, acc, sidx, didx, rows,
                 g0, g1, g2, s0, s1, s2, i0, i1, i2):
        cid = lax.axis_index("c")
        sid = lax.axis_index("s")
        ebase = (cid * _NS + sid) * _EPT
        gsem = (g0, g1, g2)
        ssem = (s0, s1, s2)
        isem = (i0, i1, i2)

        pltpu.sync_copy(zeros_h.at[pl.ds(sid * _RPT, _RPT)],
                        acc.at[pl.ds(sid * _RPT, _RPT)])
        pltpu.sync_copy(dst_h.at[pl.ds(ebase, _EPT)], didx)
        plsc.subcore_barrier()

        def fill(ci):
            b = ci % 3
            pltpu.async_copy(src_h.at[pl.ds(ebase + ci * _CH, _CH)],
                             sidx.at[b], isem[b])
            pltpu.async_copy(table_h.at[didx.at[pl.ds(ci * _CH, _CH)]],
                             rows.at[b], gsem[b])

        def wait_scatter(b):
            pltpu.make_async_copy(rows.at[b], acc.at[sidx.at[b]],
                                  ssem[b]).wait()

        fill(0)
        if _NCH > 1:
            fill(1)
        for ci in range(_NCH):
            b = ci % 3
            pltpu.make_async_copy(table_h.at[didx.at[pl.ds(ci * _CH, _CH)]],
                                  rows.at[b], gsem[b]).wait()
            pltpu.make_async_copy(src_h.at[pl.ds(ebase + ci * _CH, _CH)],
                                  sidx.at[b], isem[b]).wait()
            pltpu.async_copy(rows.at[b], acc.at[sidx.at[b]], ssem[b],
                             add=True)
            nxt = ci + 2
            if nxt < _NCH:
                bb = nxt % 3
                if ci >= 1:
                    wait_scatter(bb)
                fill(nxt)
        for k in range(max(0, _NCH - 3), _NCH):
            wait_scatter(k % 3)
        plsc.subcore_barrier()

        pltpu.sync_copy(acc.at[pl.ds(sid * _RPT, _RPT)],
                        out_h.at[cid, pl.ds(sid * _RPT, _RPT)])

    return sweep_bc


def _combine_body(p_ref, o_ref):
    o_ref[:] = p_ref[0] + p_ref[1]


def _build_combine(interpret=False):
    return pl.pallas_call(
        _combine_body,
        out_shape=jax.ShapeDtypeStruct((_N, _NG), jnp.float32),
        interpret=interpret,
    )


def _tc_body(x_ref, m1_ref, m2_ref, p3_ref, bat_ref,
             w1r_ref, w1o_ref, b1_ref, w2r_ref, w2o_ref, b2_ref,
             w3r_ref, w3o_ref, b3_ref, l1w_ref, l1b_ref, l2w_ref, l2b_ref,
             out_ref):
    def mm(a, b):
        return lax.dot_general(a, b, (((1,), (0,)), ((), ())),
                               precision=lax.Precision.HIGHEST,
                               preferred_element_type=jnp.float32)

    def tmm(a, b):
        return lax.dot_general(a, b, (((0,), (0,)), ((), ())),
                               precision=lax.Precision.HIGHEST,
                               preferred_element_type=jnp.float32)

    x = x_ref[:]
    m0t = (bat_ref[:] == lax.broadcasted_iota(jnp.int32, (_NG, _N), 0)
           ).astype(jnp.float32)
    m1 = m1_ref[:]
    m2 = m2_ref[:]
    m3 = p3_ref[0] + p3_ref[1]

    s0 = mm(m0t, x)
    s1 = tmm(m1, x)
    s2 = tmm(m2, x)
    s3 = tmm(m3, x)
    n0 = jnp.sum(m0t, axis=1, keepdims=True)
    n1 = jnp.sum(m1, axis=0, keepdims=True).reshape(_NG, 1)
    n2 = jnp.sum(m2, axis=0, keepdims=True).reshape(_NG, 1)

    w1r = w1r_ref[:]; w1o = w1o_ref[:]; b1 = b1_ref[:]
    w2r = w2r_ref[:]; w2o = w2o_ref[:]; b2 = b2_ref[:]
    w3r = w3r_ref[:]; w3o = w3o_ref[:]; b3 = b3_ref[:]

    t2 = mm(s3, w1r) + mm(s2, w1o) + n2 * b1
    t1 = mm(s2, w1r) + mm(s1, w1o) + n1 * b1
    t0 = mm(s1, w1r) + mm(s0, w1o) + n0 * b1
    a3 = mm(w2r, w3r)
    amid = mm(w2o, w3r) + mm(w2r, w3o)
    alo = mm(w2o, w3o)
    g = (mm(t2, a3) + mm(t1, amid) + mm(t0, alo)
         + n1 * mm(b2, w3r) + n0 * (mm(b2, w3o) + b3))

    z1 = jnp.maximum(mm(g, l1w_ref[:]) + l1b_ref[:], 0.0)
    z = mm(z1, l2w_ref[:]) + l2b_ref[:]
    zc = z - jnp.max(z, axis=1, keepdims=True)
    out_ref[:] = zc - jnp.log(jnp.sum(jnp.exp(zc), axis=1, keepdims=True))


def _build_tc(interpret=False):
    return pl.pallas_call(
        _tc_body,
        out_shape=jax.ShapeDtypeStruct((_NG, _NCLS), jnp.float32),
        interpret=interpret,
    )


@functools.lru_cache(maxsize=None)
def _get_kernels():
    return (_build_sweep_a(), _build_sweep_bc(), _build_combine(),
            _build_tc())


def kernel(x, edge_index, batch,
           c1_Wrel, c1_brel, c1_Wroot,
           c2_Wrel, c2_brel, c2_Wroot,
           c3_Wrel, c3_brel, c3_Wroot,
           lin1_W, lin1_b, lin2_W, lin2_b):
    ei = edge_index.astype(jnp.int32)
    bat = batch.astype(jnp.int32)
    src, dst = ei[0], ei[1]
    sweep_a, sweep_bc, combine, tc = _get_kernels()

    zflat = jnp.zeros((_N * _NG,), jnp.float32)
    p1 = sweep_a(src, dst, bat, zflat)
    m1 = combine(p1.reshape(_NC, _N, _NG))
    p2 = sweep_bc(src, dst, m1, zflat.reshape(_N, _NG))
    m2 = combine(p2)
    p3 = sweep_bc(src, dst, m2, zflat.reshape(_N, _NG))

    return tc(x, m1, m2, p3, bat.reshape(1, _N),
              c1_Wrel, c1_Wroot, c1_brel.reshape(1, _HID),
              c2_Wrel, c2_Wroot, c2_brel.reshape(1, _HID),
              c3_Wrel, c3_Wroot, c3_brel.reshape(1, _HID),
              lin1_W, lin1_b.reshape(1, _HID),
              lin2_W, lin2_b.reshape(1, _NCLS))
